# baseline (speedup 1.0000x reference)
import jax
import jax.numpy as jnp
from jax import lax
from jax.experimental import pallas as pl
from jax.experimental.pallas import tpu as pltpu
from jax.experimental.pallas import tpu_sc as plsc

B = 4096
F = 4096
NC, NS, L = 2, 16, 16
NW = NC * NS
G = 8
SL = 8
NI = F // (L * SL)
SC_ROWS = 1536
TC_ROWS = B - SC_ROWS
BRT = 256


def _row_argmax(bufs, b, r):
    iota = lax.iota(jnp.int32, L)
    neg = jnp.full((L,), -jnp.inf, jnp.float32)
    zero = jnp.zeros((L,), jnp.int32)

    def step(i, carry):
        ms = list(carry[:SL])
        us = list(carry[SL:])
        isp = jnp.full((L,), i, jnp.int32)
        for k in range(SL):
            v = bufs[b, r, pl.ds(i * (L * SL) + L * k, L)]
            nm = jnp.maximum(ms[k], v)
            us[k] = jnp.where(nm != ms[k], isp, us[k])
            ms[k] = nm
        return tuple(ms) + tuple(us)

    carry = lax.fori_loop(0, NI, step, (neg,) * SL + (zero,) * SL)
    mv = list(carry[:SL])
    mi = [carry[SL + k] * (L * SL) + (L * k) + iota for k in range(SL)]
    n = SL
    while n > 1:
        h = n // 2
        for k in range(h):
            av, ai, bv, bi = mv[k], mi[k], mv[k + h], mi[k + h]
            take_b = (bv > av) | ((bv == av) & (bi < ai))
            mv[k] = jnp.where(take_b, bv, av)
            mi[k] = jnp.where(take_b, bi, ai)
        n = h
    v, ix = mv[0], mi[0]
    for sh in (8, 4, 2, 1):
        perm = iota ^ sh
        pv = v[perm]
        pi = ix[perm]
        take_p = (pv > v) | ((pv == v) & (pi < ix))
        v = jnp.where(take_p, pv, v)
        ix = jnp.where(take_p, pi, ix)
    return ix[0]


def _make_tec_body(rows, row_off):
    rpw = rows // NW
    ng = rpw // G
    nit = ng // 2

    def _tec_body(inp, fsp, out, bufs, idxv, outv, sem0, sem1, semg):
        c = lax.axis_index("c")
        s = lax.axis_index("s")
        wid = s * NC + c
        row0 = row_off + wid * rpw
        sems = (sem0, sem1)
        iota = lax.iota(jnp.int32, L)

        def start_group(g, b):
            for r in range(G):
                pltpu.async_copy(inp.at[row0 + g * G + r, 0],
                                 bufs.at[b, r], sems[b])

        def wait_group(b):
            for r in range(G):
                pltpu.make_async_copy(inp.at[0, 0], bufs.at[b, r],
                                      sems[b]).wait()

        start_group(0, 0)

        def one_iter(it, carry):
            g0 = 2 * it
            start_group(g0 + 1, 1)
            idx = []
            wait_group(0)
            for r in range(G):
                idx.append(_row_argmax(bufs, 0, r))

            @pl.when(g0 + 2 < ng)
            def _():
                start_group(g0 + 2, 0)

            wait_group(1)
            for r in range(G):
                idx.append(_row_argmax(bufs, 1, r))
            iv = jnp.zeros((L,), jnp.int32)
            for j, sidx in enumerate(idx):
                iv = jnp.where(iota == j, sidx, iv)
            idxv[pl.ds(it * (2 * G), 2 * G)] = iv
            return carry

        lax.fori_loop(0, nit, one_iter, 0)
        pltpu.async_copy(fsp.at[idxv], outv, semg).wait()
        pltpu.sync_copy(outv, out.at[pl.ds(wid * rpw, rpw)])

    return _tec_body


def _sc_argmax_gather(input, fspace, rows, row_off):
    rpw = rows // NW
    mesh = plsc.VectorSubcoreMesh(core_axis_name="c", subcore_axis_name="s",
                                  num_cores=NC, num_subcores=NS)
    fn = pl.kernel(
        _make_tec_body(rows, row_off),
        out_type=jax.ShapeDtypeStruct((rows,), jnp.float32),
        mesh=mesh,
        compiler_params=pltpu.CompilerParams(use_tc_tiling_on_sc=True),
        scratch_types=[
            pltpu.VMEM((2, G, F), jnp.float32),
            pltpu.VMEM((rpw,), jnp.int32),
            pltpu.VMEM((rpw,), jnp.float32),
            pltpu.SemaphoreType.DMA,
            pltpu.SemaphoreType.DMA,
            pltpu.SemaphoreType.DMA,
        ],
    )
    return fn(input, fspace)


def _make_tc_body(row0):
    def _tc_body(inp_hbm, fs_ref, out_ref, buf, sems):
        i = pl.program_id(0)
        nb = pl.num_programs(0)

        def start(blk, slot):
            pltpu.make_async_copy(inp_hbm.at[pl.ds(row0 + blk * BRT, BRT), 0, :],
                                  buf.at[slot], sems.at[slot]).start()

        def wait(slot):
            pltpu.make_async_copy(inp_hbm.at[pl.ds(0, BRT), 0, :],
                                  buf.at[slot], sems.at[slot]).wait()

        @pl.when(i == 0)
        def _():
            start(0, 0)

        @pl.when(i + 1 < nb)
        def _():
            start(i + 1, (i + 1) % 2)

        wait(i % 2)
        x = buf[i % 2]
        m = jnp.max(x, axis=-1, keepdims=True)
        iota = lax.broadcasted_iota(jnp.int32, (BRT, F), 1)
        masked = jnp.where(x == m, iota, F)
        idx = jnp.min(masked, axis=-1, keepdims=True)
        onehot = (iota == idx)
        picked = jnp.where(onehot, fs_ref[...], jnp.float32(0.0))
        out_ref[...] = jnp.sum(picked, axis=-1).reshape(1, 1, BRT)

    return _tc_body


def _tc_argmax_gather(input, fspace, rows, row0):
    nbt = rows // BRT
    fs2 = fspace.reshape(1, F)
    out = pl.pallas_call(
        _make_tc_body(row0),
        grid=(nbt,),
        in_specs=[
            pl.BlockSpec(memory_space=pl.ANY),
            pl.BlockSpec((1, F), lambda i: (0, 0)),
        ],
        out_specs=pl.BlockSpec((1, 1, BRT), lambda i: (i, 0, 0)),
        out_shape=jax.ShapeDtypeStruct((nbt, 1, BRT), jnp.float32),
        scratch_shapes=[
            pltpu.VMEM((2, BRT, F), jnp.float32),
            pltpu.SemaphoreType.DMA((2,)),
        ],
    )(input, fs2)
    return out.reshape(rows)


def kernel(input, fspace):
    sc_out = _sc_argmax_gather(input, fspace, SC_ROWS, TC_ROWS)
    tc_out = _tc_argmax_gather(input, fspace, TC_ROWS, 0)
    return jnp.concatenate([tc_out, sc_out])


# speedup vs baseline: 1.0816x; 1.0816x over previous
import jax
import jax.numpy as jnp
from jax import lax
from jax.experimental import pallas as pl
from jax.experimental.pallas import tpu as pltpu
from jax.experimental.pallas import tpu_sc as plsc

B = 4096
F = 4096
NC, NS, L = 2, 16, 16
NW = NC * NS
G = 8
SL = 8
NI = F // (L * SL)
SC_ROWS = 1024
TC_ROWS = B - SC_ROWS
BRT = 512


def _row_argmax(bufs, b, r):
    iota = lax.iota(jnp.int32, L)
    neg = jnp.full((L,), -jnp.inf, jnp.float32)
    zero = jnp.zeros((L,), jnp.int32)

    def step(i, carry):
        ms = list(carry[:SL])
        us = list(carry[SL:])
        isp = jnp.full((L,), i, jnp.int32)
        for k in range(SL):
            v = bufs[b, r, pl.ds(i * (L * SL) + L * k, L)]
            nm = jnp.maximum(ms[k], v)
            us[k] = jnp.where(nm != ms[k], isp, us[k])
            ms[k] = nm
        return tuple(ms) + tuple(us)

    carry = lax.fori_loop(0, NI, step, (neg,) * SL + (zero,) * SL)
    mv = list(carry[:SL])
    mi = [carry[SL + k] * (L * SL) + (L * k) + iota for k in range(SL)]
    n = SL
    while n > 1:
        h = n // 2
        for k in range(h):
            av, ai, bv, bi = mv[k], mi[k], mv[k + h], mi[k + h]
            take_b = (bv > av) | ((bv == av) & (bi < ai))
            mv[k] = jnp.where(take_b, bv, av)
            mi[k] = jnp.where(take_b, bi, ai)
        n = h
    v, ix = mv[0], mi[0]
    for sh in (8, 4, 2, 1):
        perm = iota ^ sh
        pv = v[perm]
        pi = ix[perm]
        take_p = (pv > v) | ((pv == v) & (pi < ix))
        v = jnp.where(take_p, pv, v)
        ix = jnp.where(take_p, pi, ix)
    return ix[0]


def _make_tec_body(rows, row_off):
    rpw = rows // NW
    ng = rpw // G
    nit = ng // 2

    def _tec_body(inp, fsp, out, bufs, idxv, outv, sem0, sem1, semg):
        c = lax.axis_index("c")
        s = lax.axis_index("s")
        wid = s * NC + c
        row0 = row_off + wid * rpw
        sems = (sem0, sem1)
        iota = lax.iota(jnp.int32, L)

        def start_group(g, b):
            for r in range(G):
                pltpu.async_copy(inp.at[row0 + g * G + r, 0],
                                 bufs.at[b, r], sems[b])

        def wait_group(b):
            for r in range(G):
                pltpu.make_async_copy(inp.at[0, 0], bufs.at[b, r],
                                      sems[b]).wait()

        start_group(0, 0)

        def one_iter(it, carry):
            g0 = 2 * it
            start_group(g0 + 1, 1)
            idx = []
            wait_group(0)
            for r in range(G):
                idx.append(_row_argmax(bufs, 0, r))

            @pl.when(g0 + 2 < ng)
            def _():
                start_group(g0 + 2, 0)

            wait_group(1)
            for r in range(G):
                idx.append(_row_argmax(bufs, 1, r))
            iv = jnp.zeros((L,), jnp.int32)
            for j, sidx in enumerate(idx):
                iv = jnp.where(iota == j, sidx, iv)
            idxv[pl.ds(it * (2 * G), 2 * G)] = iv
            return carry

        lax.fori_loop(0, nit, one_iter, 0)
        pltpu.async_copy(fsp.at[idxv], outv, semg).wait()
        pltpu.sync_copy(outv, out.at[pl.ds(wid * rpw, rpw)])

    return _tec_body


def _sc_argmax_gather(input, fspace, rows, row_off):
    rpw = rows // NW
    mesh = plsc.VectorSubcoreMesh(core_axis_name="c", subcore_axis_name="s",
                                  num_cores=NC, num_subcores=NS)
    fn = pl.kernel(
        _make_tec_body(rows, row_off),
        out_type=jax.ShapeDtypeStruct((rows,), jnp.float32),
        mesh=mesh,
        compiler_params=pltpu.CompilerParams(use_tc_tiling_on_sc=True),
        scratch_types=[
            pltpu.VMEM((2, G, F), jnp.float32),
            pltpu.VMEM((rpw,), jnp.int32),
            pltpu.VMEM((rpw,), jnp.float32),
            pltpu.SemaphoreType.DMA,
            pltpu.SemaphoreType.DMA,
            pltpu.SemaphoreType.DMA,
        ],
    )
    return fn(input, fspace)


def _make_tc_body(row0):
    def _tc_body(inp_hbm, fs_ref, out_ref, buf, sems):
        i = pl.program_id(0)
        nb = pl.num_programs(0)

        def start(blk, slot):
            pltpu.make_async_copy(inp_hbm.at[pl.ds(row0 + blk * BRT, BRT), 0, :],
                                  buf.at[slot], sems.at[slot]).start()

        def wait(slot):
            pltpu.make_async_copy(inp_hbm.at[pl.ds(0, BRT), 0, :],
                                  buf.at[slot], sems.at[slot]).wait()

        @pl.when(i == 0)
        def _():
            start(0, 0)

        @pl.when(i + 1 < nb)
        def _():
            start(i + 1, (i + 1) % 2)

        wait(i % 2)
        x = buf[i % 2]
        m = jnp.max(x, axis=-1, keepdims=True)
        iota = lax.broadcasted_iota(jnp.int32, (BRT, F), 1)
        masked = jnp.where(x == m, iota, F)
        idx = jnp.min(masked, axis=-1, keepdims=True)
        onehot = (iota == idx)
        picked = jnp.where(onehot, fs_ref[...], jnp.float32(0.0))
        out_ref[...] = jnp.sum(picked, axis=-1).reshape(1, 1, BRT)

    return _tc_body


def _tc_argmax_gather(input, fspace, rows, row0):
    nbt = rows // BRT
    fs2 = fspace.reshape(1, F)
    out = pl.pallas_call(
        _make_tc_body(row0),
        grid=(nbt,),
        in_specs=[
            pl.BlockSpec(memory_space=pl.ANY),
            pl.BlockSpec((1, F), lambda i: (0, 0)),
        ],
        out_specs=pl.BlockSpec((1, 1, BRT), lambda i: (i, 0, 0)),
        out_shape=jax.ShapeDtypeStruct((nbt, 1, BRT), jnp.float32),
        scratch_shapes=[
            pltpu.VMEM((2, BRT, F), jnp.float32),
            pltpu.SemaphoreType.DMA((2,)),
        ],
    )(input, fs2)
    return out.reshape(rows)


def kernel(input, fspace):
    sc_out = _sc_argmax_gather(input, fspace, SC_ROWS, TC_ROWS)
    tc_out = _tc_argmax_gather(input, fspace, TC_ROWS, 0)
    return jnp.concatenate([tc_out, sc_out])
